# Initial kernel scaffold; baseline (speedup 1.0000x reference)
import functools

import numpy as np
import jax
import jax.numpy as jnp
from jax import lax
from jax.experimental import pallas as pl
from jax.experimental.pallas import tpu as pltpu
from jax.experimental.pallas import tpu_sc as plsc

_GAMMA = 12.0
_EMB_INIT = 0.109375
_PHASE_DIV = np.float32(_EMB_INIT / np.pi)

_L = 16
_CHUNK = 64


def _cossin_body(rel_ref, cs_ref):
    phase = rel_ref[...] / _PHASE_DIV
    cs_ref[:, 0:128] = jnp.cos(phase)
    cs_ref[:, 128:256] = jnp.sin(phase)


def _make_cossin(rel_table):
    n_rels, d_rel = rel_table.shape
    return pl.pallas_call(
        _cossin_body,
        out_shape=jax.ShapeDtypeStruct((n_rels, 2 * d_rel), jnp.float32),
    )(rel_table)


def _sqrt16(s2):
    s2 = jnp.maximum(s2, jnp.float32(1e-12))
    i = plsc.bitcast(s2, jnp.int32)
    i = jnp.int32(0x5F3759DF) - (i >> 1)
    y = plsc.bitcast(i, jnp.float32)
    h = s2 * jnp.float32(0.5)
    y = y * (jnp.float32(1.5) - h * y * y)
    y = y * (jnp.float32(1.5) - h * y * y)
    return s2 * y


def _make_sc_kernel(n_nodes, d_feat, n_rels, e_pad):
    info = plsc.get_sparse_core_info()
    nc, ns = info.num_cores, info.num_subcores
    nw = nc * ns
    per_tile = e_pad // nw
    chunks = per_tile // _CHUNK
    d_rel = d_feat // 2

    mesh = plsc.VectorSubcoreMesh(core_axis_name="c", subcore_axis_name="s")

    @functools.partial(
        pl.kernel,
        out_type=jax.ShapeDtypeStruct((e_pad,), jnp.float32),
        mesh=mesh,
        scratch_types=[
            pltpu.VMEM((per_tile,), jnp.int32),
            pltpu.VMEM((per_tile,), jnp.int32),
            pltpu.VMEM((per_tile,), jnp.int32),
            pltpu.VMEM((_CHUNK, d_feat), jnp.float32),
            pltpu.VMEM((_CHUNK, d_feat), jnp.float32),
            pltpu.VMEM((_CHUNK, d_feat), jnp.float32),
            pltpu.VMEM((_L * _L,), jnp.float32),
            pltpu.VMEM((_CHUNK,), jnp.float32),
            pltpu.SemaphoreType.DMA,
            pltpu.SemaphoreType.DMA,
            pltpu.SemaphoreType.DMA,
        ],
    )
    def sc_kernel(node_hbm, cs_hbm, hidx_hbm, tidx_hbm, ridx_hbm, out_hbm,
                  hidx_v, tidx_v, ridx_v, headb, tailb, csb, accb, outb,
                  sem_h, sem_t, sem_r):
        wid = lax.axis_index("s") * nc + lax.axis_index("c")
        base = wid * per_tile
        pltpu.sync_copy(hidx_hbm.at[pl.ds(base, per_tile)], hidx_v)
        pltpu.sync_copy(tidx_hbm.at[pl.ds(base, per_tile)], tidx_v)
        pltpu.sync_copy(ridx_hbm.at[pl.ds(base, per_tile)], ridx_v)

        col0 = lax.iota(jnp.int32, _L) * _L

        def chunk_body(ci, carry):
            coff = ci * _CHUNK
            cp_h = pltpu.async_copy(
                node_hbm.at[hidx_v.at[pl.ds(coff, _CHUNK)]], headb, sem_h)
            cp_t = pltpu.async_copy(
                node_hbm.at[tidx_v.at[pl.ds(coff, _CHUNK)]], tailb, sem_t)
            cp_r = pltpu.async_copy(
                cs_hbm.at[ridx_v.at[pl.ds(coff, _CHUNK)]], csb, sem_r)
            cp_h.wait()
            cp_t.wait()
            cp_r.wait()

            def group_body(g, carry2):
                for e in range(_L):
                    edge = g * _L + e
                    acc = jnp.zeros((_L,), jnp.float32)
                    for si in range(d_rel // _L):
                        off = si * _L
                        rh = headb[edge, pl.ds(off, _L)]
                        ih = headb[edge, pl.ds(d_rel + off, _L)]
                        rt = tailb[edge, pl.ds(off, _L)]
                        it = tailb[edge, pl.ds(d_rel + off, _L)]
                        cv = csb[edge, pl.ds(off, _L)]
                        sv = csb[edge, pl.ds(d_rel + off, _L)]
                        a = rh * cv - ih * sv - rt
                        b = rh * sv + ih * cv - it
                        acc = acc + _sqrt16(a * a + b * b)
                    accb[pl.ds(e * _L, _L)] = acc
                tot = plsc.load_gather(accb, [col0])
                for j in range(1, _L):
                    tot = tot + plsc.load_gather(accb, [col0 + j])
                outb[pl.ds(g * _L, _L)] = jnp.float32(_GAMMA) - tot
                return carry2

            lax.fori_loop(0, _CHUNK // _L, group_body, 0)
            pltpu.sync_copy(outb, out_hbm.at[pl.ds(base + coff, _CHUNK)])
            return carry

        lax.fori_loop(0, chunks, chunk_body, 0)

    return sc_kernel


def kernel(node_emb, rel_table, edge_index, rel_ids):
    n_nodes, d_feat = node_emb.shape
    n_rels, d_rel = rel_table.shape
    e = edge_index.shape[1]

    info = plsc.get_sparse_core_info()
    nw = info.num_cores * info.num_subcores
    gran = nw * _CHUNK
    e_pad = ((e + gran - 1) // gran) * gran

    cs = _make_cossin(rel_table)

    pad = e_pad - e
    hidx = jnp.concatenate([edge_index[0], jnp.zeros((pad,), jnp.int32)])
    tidx = jnp.concatenate([edge_index[1], jnp.zeros((pad,), jnp.int32)])
    ridx = jnp.concatenate([rel_ids, jnp.zeros((pad,), jnp.int32)])

    sck = _make_sc_kernel(n_nodes, d_feat, n_rels, e_pad)
    out = sck(node_emb, cs, hidx, tidx, ridx)
    return out[:e]


# speedup vs baseline: 2.1390x; 2.1390x over previous
import functools

import numpy as np
import jax
import jax.numpy as jnp
from jax import lax
from jax.experimental import pallas as pl
from jax.experimental.pallas import tpu as pltpu
from jax.experimental.pallas import tpu_sc as plsc

_GAMMA = 12.0
_EMB_INIT = 0.109375
_PHASE_DIV = np.float32(_EMB_INIT / np.pi)

_L = 16
_CHUNK = 64


def _cossin_body(rel_ref, cs_ref):
    phase = rel_ref[...] / _PHASE_DIV
    cs_ref[:, 0:128] = jnp.cos(phase)
    cs_ref[:, 128:256] = jnp.sin(phase)


def _make_cossin(rel_table):
    n_rels, d_rel = rel_table.shape
    return pl.pallas_call(
        _cossin_body,
        out_shape=jax.ShapeDtypeStruct((n_rels, 2 * d_rel), jnp.float32),
    )(rel_table)


def _sqrt16(s2):
    s2 = jnp.maximum(s2, jnp.float32(1e-12))
    i = lax.bitcast_convert_type(s2, jnp.int32)
    i = jnp.int32(0x5F3759DF) - (i >> 1)
    y = lax.bitcast_convert_type(i, jnp.float32)
    h = s2 * jnp.float32(0.5)
    y = y * (jnp.float32(1.5) - h * y * y)
    y = y * (jnp.float32(1.5) - h * y * y)
    return s2 * y


def _make_sc_kernel(n_nodes, d_feat, n_rels, e_pad):
    info = plsc.get_sparse_core_info()
    nc, ns = info.num_cores, info.num_subcores
    nw = nc * ns
    per_tile = e_pad // nw
    chunks = per_tile // _CHUNK
    d_rel = d_feat // 2

    mesh = plsc.VectorSubcoreMesh(core_axis_name="c", subcore_axis_name="s")

    @functools.partial(
        pl.kernel,
        out_type=jax.ShapeDtypeStruct((e_pad,), jnp.float32),
        mesh=mesh,
        scratch_types=[
            pltpu.VMEM((per_tile,), jnp.int32),
            pltpu.VMEM((per_tile,), jnp.int32),
            pltpu.VMEM((per_tile,), jnp.int32),
            pltpu.VMEM((_CHUNK, d_feat), jnp.float32),
            pltpu.VMEM((_CHUNK, d_feat), jnp.float32),
            pltpu.VMEM((_CHUNK, d_feat), jnp.float32),
            pltpu.VMEM((_L * _L,), jnp.float32),
            pltpu.VMEM((_CHUNK,), jnp.float32),
            pltpu.SemaphoreType.DMA,
            pltpu.SemaphoreType.DMA,
            pltpu.SemaphoreType.DMA,
        ],
        compiler_params=pltpu.CompilerParams(needs_layout_passes=False),
    )
    def sc_kernel(node_hbm, cs_hbm, hidx_hbm, tidx_hbm, ridx_hbm, out_hbm,
                  hidx_v, tidx_v, ridx_v, headb, tailb, csb, accb, outb,
                  sem_h, sem_t, sem_r):
        wid = lax.axis_index("s") * nc + lax.axis_index("c")
        base = wid * per_tile
        pltpu.sync_copy(hidx_hbm.at[pl.ds(base, per_tile)], hidx_v)
        pltpu.sync_copy(tidx_hbm.at[pl.ds(base, per_tile)], tidx_v)
        pltpu.sync_copy(ridx_hbm.at[pl.ds(base, per_tile)], ridx_v)

        col0 = lax.iota(jnp.int32, _L) * _L

        def chunk_body(ci, carry):
            coff = ci * _CHUNK
            cp_h = pltpu.async_copy(
                node_hbm.at[hidx_v.at[pl.ds(coff, _CHUNK)]], headb, sem_h)
            cp_t = pltpu.async_copy(
                node_hbm.at[tidx_v.at[pl.ds(coff, _CHUNK)]], tailb, sem_t)
            cp_r = pltpu.async_copy(
                cs_hbm.at[ridx_v.at[pl.ds(coff, _CHUNK)]], csb, sem_r)
            cp_h.wait()
            cp_t.wait()
            cp_r.wait()

            def group_body(g, carry2):
                for e in range(_L):
                    edge = g * _L + e
                    acc = jnp.zeros((_L,), jnp.float32)
                    for si in range(d_rel // _L):
                        off = si * _L
                        rh = headb[edge, pl.ds(off, _L)]
                        ih = headb[edge, pl.ds(d_rel + off, _L)]
                        rt = tailb[edge, pl.ds(off, _L)]
                        it = tailb[edge, pl.ds(d_rel + off, _L)]
                        cv = csb[edge, pl.ds(off, _L)]
                        sv = csb[edge, pl.ds(d_rel + off, _L)]
                        a = rh * cv - ih * sv - rt
                        b = rh * sv + ih * cv - it
                        acc = acc + _sqrt16(a * a + b * b)
                    accb[pl.ds(e * _L, _L)] = acc
                tot = plsc.load_gather(accb, [col0])
                for j in range(1, _L):
                    tot = tot + plsc.load_gather(accb, [col0 + j])
                outb[pl.ds(g * _L, _L)] = jnp.float32(_GAMMA) - tot
                return carry2

            lax.fori_loop(0, _CHUNK // _L, group_body, 0)
            pltpu.sync_copy(outb, out_hbm.at[pl.ds(base + coff, _CHUNK)])
            return carry

        lax.fori_loop(0, chunks, chunk_body, 0)

    return sc_kernel


def kernel(node_emb, rel_table, edge_index, rel_ids):
    n_nodes, d_feat = node_emb.shape
    n_rels, d_rel = rel_table.shape
    e = edge_index.shape[1]

    info = plsc.get_sparse_core_info()
    nw = info.num_cores * info.num_subcores
    gran = nw * _CHUNK
    e_pad = ((e + gran - 1) // gran) * gran

    cs = _make_cossin(rel_table)

    pad = e_pad - e
    hidx = jnp.concatenate([edge_index[0], jnp.zeros((pad,), jnp.int32)])
    tidx = jnp.concatenate([edge_index[1], jnp.zeros((pad,), jnp.int32)])
    ridx = jnp.concatenate([rel_ids, jnp.zeros((pad,), jnp.int32)])

    sck = _make_sc_kernel(n_nodes, d_feat, n_rels, e_pad)
    out = sck(node_emb, cs, hidx, tidx, ridx)
    return out[:e]
